# baseline (speedup 1.0000x reference)
import jax
import jax.numpy as jnp
from jax.experimental import pallas as pl
from jax.experimental.pallas import tpu as pltpu

B = 1024
D = 512
C = 1000
K = 5
BLK = 128
NBLK = B // BLK


def _body(feat_hbm, p_hbm, r_hbm, out_ref,
          feat_v, p_v, r_v, keys_v, sem_f, sem_p, sem_r):
    cp_f = pltpu.make_async_copy(feat_hbm, feat_v, sem_f)
    cp_p = pltpu.make_async_copy(p_hbm, p_v, sem_p)
    cp_r = pltpu.make_async_copy(r_hbm, r_v, sem_r)
    cp_f.start()
    cp_p.start()
    cp_r.start()

    cp_f.wait()
    x = feat_v[:]
    iota = jax.lax.broadcasted_iota(jnp.int32, (B, D), 1)
    idxs = []
    for _ in range(K):
        m = jnp.max(x, axis=1, keepdims=True)
        idx = jnp.min(jnp.where(x == m, iota, D), axis=1)
        idxs.append(idx)
        x = jnp.where(iota == idx[:, None], -jnp.inf, x)
    a = (idxs[0] * D + idxs[1]) * D + idxs[2]
    b = idxs[3] * D + idxs[4]
    keys_v[:] = jnp.concatenate(
        [a[None, :], b[None, :], jnp.zeros((6, B), jnp.int32)], axis=0)

    cp_p.wait()
    cp_r.wait()

    ka = keys_v[0:1, :]
    kb = keys_v[1:2, :]
    HBB = B // 8
    acc = jnp.zeros((1, 1), jnp.float32)
    for k in range(8):
        prod = jax.lax.dot_general(
            p_v[k * HBB:(k + 1) * HBB, :], r_v[:],
            (((1,), (1,)), ((), ())),
            preferred_element_type=jnp.float32)
        my_a = jnp.reshape(ka[:, k * HBB:(k + 1) * HBB], (HBB, 1))
        my_b = jnp.reshape(kb[:, k * HBB:(k + 1) * HBB], (HBB, 1))
        simb = (my_a == ka) & (my_b == kb)
        arg = jnp.where(simb, prod, 1.0 - prod)
        loss = jnp.maximum(jnp.log(arg), -100.0)
        acc += jnp.full((1, 1), -1.0 / (B * B)) * jnp.sum(loss)
    out_ref[:, :] = acc


@jax.jit
def kernel(unlabel_feat, unlabel_prob, rot_unlabel_prob):
    out = pl.pallas_call(
        _body,
        grid=(1,),
        in_specs=[
            pl.BlockSpec(memory_space=pl.ANY),
            pl.BlockSpec(memory_space=pl.ANY),
            pl.BlockSpec(memory_space=pl.ANY),
        ],
        out_specs=pl.BlockSpec((1, 1), lambda i: (0, 0)),
        out_shape=jax.ShapeDtypeStruct((1, 1), jnp.float32),
        scratch_shapes=[
            pltpu.VMEM((B, D), jnp.float32),
            pltpu.VMEM((B, C), jnp.float32),
            pltpu.VMEM((B, C), jnp.float32),
            pltpu.VMEM((8, B), jnp.int32),
            pltpu.SemaphoreType.DMA,
            pltpu.SemaphoreType.DMA,
            pltpu.SemaphoreType.DMA,
        ],
    )(unlabel_feat, unlabel_prob, rot_unlabel_prob)
    return out[0, 0]


# speedup vs baseline: 1.1790x; 1.1790x over previous
import jax
import jax.numpy as jnp
from jax.experimental import pallas as pl
from jax.experimental.pallas import tpu as pltpu

B = 1024
D = 512
C = 1000
K = 5
BLK = 128
NBLK = B // BLK


def _body(feat_hbm, p_hbm, r_hbm, out_ref,
          feat_v, p_v, r_v, keys_v, sem_f, sem_p, sem_r):
    cp_f = pltpu.make_async_copy(feat_hbm, feat_v, sem_f)
    cp_p = pltpu.make_async_copy(p_hbm, p_v, sem_p)
    cp_r = pltpu.make_async_copy(r_hbm, r_v, sem_r)
    cp_f.start()
    cp_p.start()
    cp_r.start()

    cp_f.wait()
    x = feat_v[:]
    iota = jax.lax.broadcasted_iota(jnp.int32, (B, D), 1)
    idxs = []
    for _ in range(K):
        m = jnp.max(x, axis=1, keepdims=True)
        idx = jnp.min(jnp.where(x == m, iota, D), axis=1)
        idxs.append(idx)
        x = jnp.where(iota == idx[:, None], -jnp.inf, x)
    a = (idxs[0] * D + idxs[1]) * D + idxs[2]
    b = idxs[3] * D + idxs[4]
    keys_v[:] = jnp.concatenate(
        [a[None, :], b[None, :], jnp.zeros((6, B), jnp.int32)], axis=0)

    cp_p.wait()
    cp_r.wait()

    ka = keys_v[0:1, :]
    kb = keys_v[1:2, :]
    HBB = B // 4
    acc = jnp.zeros((1, 1), jnp.float32)
    for k in range(4):
        prod = jax.lax.dot_general(
            p_v[k * HBB:(k + 1) * HBB, :], r_v[:],
            (((1,), (1,)), ((), ())),
            preferred_element_type=jnp.float32)
        my_a = jnp.reshape(ka[:, k * HBB:(k + 1) * HBB], (HBB, 1))
        my_b = jnp.reshape(kb[:, k * HBB:(k + 1) * HBB], (HBB, 1))
        simb = (my_a == ka) & (my_b == kb)
        arg = jnp.where(simb, prod, 1.0 - prod)
        loss = jnp.maximum(jnp.log(arg), -100.0)
        acc += jnp.full((1, 1), -1.0 / (B * B)) * jnp.sum(loss)
    out_ref[:, :] = acc


@jax.jit
def kernel(unlabel_feat, unlabel_prob, rot_unlabel_prob):
    out = pl.pallas_call(
        _body,
        grid=(1,),
        in_specs=[
            pl.BlockSpec(memory_space=pl.ANY),
            pl.BlockSpec(memory_space=pl.ANY),
            pl.BlockSpec(memory_space=pl.ANY),
        ],
        out_specs=pl.BlockSpec((1, 1), lambda i: (0, 0)),
        out_shape=jax.ShapeDtypeStruct((1, 1), jnp.float32),
        scratch_shapes=[
            pltpu.VMEM((B, D), jnp.float32),
            pltpu.VMEM((B, C), jnp.float32),
            pltpu.VMEM((B, C), jnp.float32),
            pltpu.VMEM((8, B), jnp.int32),
            pltpu.SemaphoreType.DMA,
            pltpu.SemaphoreType.DMA,
            pltpu.SemaphoreType.DMA,
        ],
    )(unlabel_feat, unlabel_prob, rot_unlabel_prob)
    return out[0, 0]


# speedup vs baseline: 1.1857x; 1.0057x over previous
import jax
import jax.numpy as jnp
from jax.experimental import pallas as pl
from jax.experimental.pallas import tpu as pltpu

B = 1024
D = 512
C = 1000
K = 5


def _body(feat_hbm, p_hbm, r_hbm, out_ref,
          feat_v, p_v, r_v, keys_v, sem_f, sem_p, sem_r):
    cp_f = pltpu.make_async_copy(feat_hbm, feat_v, sem_f)
    cp_p = pltpu.make_async_copy(p_hbm, p_v, sem_p)
    cp_r = pltpu.make_async_copy(r_hbm, r_v, sem_r)
    cp_f.start()
    cp_p.start()
    cp_r.start()

    cp_f.wait()
    x = feat_v[:]
    iota = jax.lax.broadcasted_iota(jnp.int32, (B, D), 1)
    idxs = []
    for _ in range(K):
        m = jnp.max(x, axis=1, keepdims=True)
        idx = jnp.min(jnp.where(x == m, iota, D), axis=1)
        idxs.append(idx)
        x = jnp.where(iota == idx[:, None], -jnp.inf, x)
    a = (idxs[0] * D + idxs[1]) * D + idxs[2]
    b = idxs[3] * D + idxs[4]
    keys_v[:] = jnp.concatenate(
        [a[None, :], b[None, :], jnp.zeros((6, B), jnp.int32)], axis=0)

    cp_p.wait()
    cp_r.wait()

    ka = keys_v[0:1, :]
    kb = keys_v[1:2, :]
    HBB = B // 4
    r_all = r_v[:]
    acc = jnp.zeros((1, 1), jnp.float32)
    for k in range(4):
        prod = jax.lax.dot_general(
            p_v[k * HBB:(k + 1) * HBB, :], r_all,
            (((1,), (1,)), ((), ())),
            preferred_element_type=jnp.float32)
        my_a = jnp.reshape(ka[:, k * HBB:(k + 1) * HBB], (HBB, 1))
        my_b = jnp.reshape(kb[:, k * HBB:(k + 1) * HBB], (HBB, 1))
        simb = (my_a == ka) & (my_b == kb)
        arg = jnp.where(simb, prod, 1.0 - prod)
        loss = jnp.maximum(jnp.log(arg), -100.0)
        acc += jnp.full((1, 1), -1.0 / (B * B)) * jnp.sum(loss)
    out_ref[:, :] = acc


@jax.jit
def kernel(unlabel_feat, unlabel_prob, rot_unlabel_prob):
    out = pl.pallas_call(
        _body,
        grid=(1,),
        in_specs=[
            pl.BlockSpec(memory_space=pl.ANY),
            pl.BlockSpec(memory_space=pl.ANY),
            pl.BlockSpec(memory_space=pl.ANY),
        ],
        out_specs=pl.BlockSpec((1, 1), lambda i: (0, 0)),
        out_shape=jax.ShapeDtypeStruct((1, 1), jnp.float32),
        scratch_shapes=[
            pltpu.VMEM((B, D), jnp.float32),
            pltpu.VMEM((B, C), jnp.float32),
            pltpu.VMEM((B, C), jnp.float32),
            pltpu.VMEM((8, B), jnp.int32),
            pltpu.SemaphoreType.DMA,
            pltpu.SemaphoreType.DMA,
            pltpu.SemaphoreType.DMA,
        ],
    )(unlabel_feat, unlabel_prob, rot_unlabel_prob)
    return out[0, 0]
